# baseline (speedup 1.0000x reference)
import functools

import jax
import jax.numpy as jnp
from jax import lax
from jax.experimental import pallas as pl
from jax.experimental.pallas import tpu as pltpu
from jax.experimental.pallas import tpu_sc as plsc

_N = 100000
_E = 6400000
_D = 4
_H = 16
_G = 256

_DP = 8
_NC = 2
_NS = 16
_NW = _NC * _NS
_LANE = 128
_K = 20
_ROWS = _E // _LANE
_CHUNKS = _ROWS // _K
_ITERS = 80
_NPT = _N // _NS


def _agg_body(xz_hbm, e_hbm, out_hbm,
              agg_sh, sidx, didx, rows, isem, gsem, ssem):
    c = lax.axis_index("c")
    s = lax.axis_index("s")
    wid = c * _NS + s
    nbase = s * _NPT

    pltpu.sync_copy(xz_hbm.at[pl.ds(_N + nbase, _NPT)],
                    agg_sh.at[pl.ds(nbase, _NPT)])

    _CE = _K * _LANE

    def load_idx(b, m):
        e0 = m * _CE
        pltpu.async_copy(e_hbm.at[0, pl.ds(e0, _CE)], sidx.at[b], isem)
        pltpu.async_copy(e_hbm.at[1, pl.ds(e0, _CE)], didx.at[b], isem)

    def wait_idx(b):
        pltpu.make_async_copy(e_hbm.at[0, pl.ds(0, _CE)], sidx.at[b],
                              isem).wait()
        pltpu.make_async_copy(e_hbm.at[1, pl.ds(0, _CE)], didx.at[b],
                              isem).wait()

    def fire_gathers(b):
        for j in range(_K):
            pltpu.async_copy(
                xz_hbm.at[sidx.at[b].at[pl.ds(j * _LANE, _LANE)]],
                rows.at[b].at[j], gsem)

    def drain_gathers(b):
        for j in range(_K):
            pltpu.make_async_copy(
                xz_hbm.at[sidx.at[b].at[pl.ds(j * _LANE, _LANE)]],
                rows.at[b].at[j], gsem).wait()

    def fire_scatters(b):
        for j in range(_K):
            pltpu.async_copy(
                rows.at[b].at[j],
                agg_sh.at[didx.at[b].at[pl.ds(j * _LANE, _LANE)]],
                ssem, add=True)

    def drain_scatters(b):
        for j in range(_K):
            pltpu.make_async_copy(
                rows.at[b].at[j],
                agg_sh.at[didx.at[b].at[pl.ds(j * _LANE, _LANE)]],
                ssem).wait()

    plsc.subcore_barrier()
    load_idx(0, wid)

    def pair(go, carry):
        for b in (0, 1):
            t = 2 * go + b
            m = t * _NW + wid

            @pl.when(m < _CHUNKS)
            def _gather():
                wait_idx(b)
                fire_gathers(b)

            @pl.when((t >= 1) & (m - _NW < _CHUNKS))
            def _drain_prev():
                drain_scatters(1 - b)

            @pl.when(m + _NW < _CHUNKS)
            def _prefetch():
                load_idx(1 - b, m + _NW)

            @pl.when(m < _CHUNKS)
            def _scatter():
                drain_gathers(b)
                fire_scatters(b)

        return carry

    lax.fori_loop(0, _ITERS // 2, pair, 0)

    @pl.when((_ITERS - 1) * _NW + wid < _CHUNKS)
    def _tail():
        drain_scatters((_ITERS - 1) % 2)

    plsc.subcore_barrier()
    pltpu.sync_copy(agg_sh.at[pl.ds(nbase, _NPT)],
                    out_hbm.at[pl.ds(c * _N + nbase, _NPT)])


_agg_kernel = functools.partial(
    pl.kernel,
    out_type=jax.ShapeDtypeStruct((2 * _N, _DP), jnp.float32),
    mesh=plsc.VectorSubcoreMesh(core_axis_name="c", subcore_axis_name="s"),
    compiler_params=pltpu.CompilerParams(use_tc_tiling_on_sc=False),
    scratch_types=[
        pltpu.VMEM_SHARED((_N, _DP), jnp.float32),
        pltpu.VMEM((2, _K * _LANE), jnp.int32),
        pltpu.VMEM((2, _K * _LANE), jnp.int32),
        pltpu.VMEM((2, _K, _LANE, _DP), jnp.float32),
        pltpu.SemaphoreType.DMA,
        pltpu.SemaphoreType.DMA,
        pltpu.SemaphoreType.DMA,
    ],
)(_agg_body)


_R = 5000
_GRID = _N // _R


def _mlp_pool_body(eps_sm, x_ref, agg_any, b_ref,
                   w1_ref, b1_ref, w2_ref, b2_ref, out_ref, acc_s, acc_c,
                   a0_v, a1_v, dsem):
    i = pl.program_id(0)

    @pl.when(i == 0)
    def _init():
        acc_s[...] = jnp.zeros_like(acc_s)
        acc_c[...] = jnp.zeros_like(acc_c)

    cp0 = pltpu.make_async_copy(agg_any.at[pl.ds(i * _R, _R)], a0_v, dsem)
    cp1 = pltpu.make_async_copy(agg_any.at[pl.ds(_N + i * _R, _R)], a1_v,
                                dsem)
    cp0.start()
    cp1.start()
    cp0.wait()
    cp1.wait()
    h = ((1.0 + eps_sm[0]) * x_ref[...]
         + a0_v[:, 0:_D] + a1_v[:, 0:_D])
    h = jnp.dot(h, w1_ref[...], preferred_element_type=jnp.float32)
    h = jnp.maximum(h + b1_ref[...], 0.0)
    h = jnp.dot(h, w2_ref[...], preferred_element_type=jnp.float32)
    h = jnp.maximum(h + b2_ref[...], 0.0)

    seg = b_ref[0, 0, :]
    oh = (seg[:, None] ==
          lax.broadcasted_iota(jnp.int32, (_R, _G), 1)).astype(jnp.float32)
    acc_s[...] += lax.dot_general(oh, h, (((0,), (0,)), ((), ())),
                                  preferred_element_type=jnp.float32)
    acc_c[...] += jnp.sum(oh, axis=0, keepdims=True)

    @pl.when(i == _GRID - 1)
    def _fin():
        cnt = jnp.maximum(acc_c[0, :], 1.0)
        pooled = acc_s[...] / cnt[:, None]
        m = jnp.max(pooled, axis=1, keepdims=True)
        e = jnp.exp(pooled - m)
        lse = jnp.log(jnp.sum(e, axis=1, keepdims=True))
        out_ref[...] = pooled - m - lse


def _mlp_pool(x, aggflat, batch, eps, W1, b1, W2, b2):
    batch3 = batch.reshape(_GRID, 1, _R)
    return pl.pallas_call(
        _mlp_pool_body,
        grid=(_GRID,),
        in_specs=[
            pl.BlockSpec(memory_space=pltpu.SMEM),
            pl.BlockSpec((_R, _D), lambda i: (i, 0)),
            pl.BlockSpec(memory_space=pl.ANY),
            pl.BlockSpec((1, 1, _R), lambda i: (i, 0, 0)),
            pl.BlockSpec((_D, _H), lambda i: (0, 0)),
            pl.BlockSpec((1, _H), lambda i: (0, 0)),
            pl.BlockSpec((_H, _D), lambda i: (0, 0)),
            pl.BlockSpec((1, _D), lambda i: (0, 0)),
        ],
        out_specs=pl.BlockSpec((_G, _D), lambda i: (0, 0)),
        out_shape=jax.ShapeDtypeStruct((_G, _D), jnp.float32),
        scratch_shapes=[
            pltpu.VMEM((_G, _D), jnp.float32),
            pltpu.VMEM((1, _G), jnp.float32),
            pltpu.VMEM((_R, _DP), jnp.float32),
            pltpu.VMEM((_R, _DP), jnp.float32),
            pltpu.SemaphoreType.DMA,
        ],
    )(eps.reshape(1), x, aggflat, batch3,
      W1, b1.reshape(1, _H), W2, b2.reshape(1, _D))


def kernel(x, edge_index, batch, eps, W1, b1, W2, b2):
    xz = jnp.pad(x, ((0, _N), (0, _DP - _D)))
    aggflat = _agg_kernel(xz, edge_index)
    return _mlp_pool(x, aggflat, batch, eps, W1, b1, W2, b2)


# speedup vs baseline: 1.0849x; 1.0849x over previous
import functools

import jax
import jax.numpy as jnp
from jax import lax
from jax.experimental import pallas as pl
from jax.experimental.pallas import tpu as pltpu
from jax.experimental.pallas import tpu_sc as plsc

_N = 100000
_E = 6400000
_D = 4
_H = 16
_G = 256

_DP = 8
_NC = 2
_NS = 16
_NW = _NC * _NS
_LANE = 128
_K = 20
_ROWS = _E // _LANE
_CHUNKS = _ROWS // _K
_ITERS = 80
_NPT = _N // _NS


def _agg_body(xz_hbm, e_hbm, out_hbm,
              agg_sh, sidx, didx, rows, isem, gsem, ssem):
    c = lax.axis_index("c")
    s = lax.axis_index("s")
    wid = c * _NS + s
    nbase = s * _NPT

    pltpu.sync_copy(xz_hbm.at[pl.ds(_N + nbase, _NPT)],
                    agg_sh.at[pl.ds(nbase, _NPT)])

    _CE = _K * _LANE

    def load_idx(b, m):
        e0 = m * _CE
        pltpu.async_copy(e_hbm.at[0, pl.ds(e0, _CE)], sidx.at[b], isem)
        pltpu.async_copy(e_hbm.at[1, pl.ds(e0, _CE)], didx.at[b], isem)

    def wait_idx(b):
        pltpu.make_async_copy(e_hbm.at[0, pl.ds(0, _CE)], sidx.at[b],
                              isem).wait()
        pltpu.make_async_copy(e_hbm.at[1, pl.ds(0, _CE)], didx.at[b],
                              isem).wait()

    def fire_gathers(b):
        for j in range(_K):
            pltpu.async_copy(
                xz_hbm.at[sidx.at[b].at[pl.ds(j * _LANE, _LANE)]],
                rows.at[b].at[j], gsem)

    def drain_gathers(b):
        for j in range(_K):
            pltpu.make_async_copy(
                xz_hbm.at[sidx.at[b].at[pl.ds(j * _LANE, _LANE)]],
                rows.at[b].at[j], gsem).wait()

    def fire_scatters(b):
        for j in range(_K):
            pltpu.async_copy(
                rows.at[b].at[j],
                agg_sh.at[didx.at[b].at[pl.ds(j * _LANE, _LANE)]],
                ssem, add=True)

    def drain_scatters(b):
        for j in range(_K):
            pltpu.make_async_copy(
                rows.at[b].at[j],
                agg_sh.at[didx.at[b].at[pl.ds(j * _LANE, _LANE)]],
                ssem).wait()

    plsc.subcore_barrier()
    load_idx(0, wid)

    def pair(go, carry):
        for b in (0, 1):
            t = 2 * go + b
            m = t * _NW + wid

            @pl.when(m < _CHUNKS)
            def _gather():
                wait_idx(b)
                fire_gathers(b)

            @pl.when((t >= 1) & (m - _NW < _CHUNKS))
            def _drain_prev():
                drain_scatters(1 - b)

            @pl.when(m + _NW < _CHUNKS)
            def _prefetch():
                load_idx(1 - b, m + _NW)

            @pl.when(m < _CHUNKS)
            def _scatter():
                drain_gathers(b)
                fire_scatters(b)

        return carry

    lax.fori_loop(0, _ITERS // 2, pair, 0)

    @pl.when((_ITERS - 1) * _NW + wid < _CHUNKS)
    def _tail():
        drain_scatters((_ITERS - 1) % 2)

    plsc.subcore_barrier()
    pltpu.sync_copy(agg_sh.at[pl.ds(nbase, _NPT)],
                    out_hbm.at[pl.ds(c * _N + nbase, _NPT)])


_agg_kernel = functools.partial(
    pl.kernel,
    out_type=jax.ShapeDtypeStruct((2 * _N, _DP), jnp.float32),
    mesh=plsc.VectorSubcoreMesh(core_axis_name="c", subcore_axis_name="s"),
    compiler_params=pltpu.CompilerParams(use_tc_tiling_on_sc=False),
    scratch_types=[
        pltpu.VMEM_SHARED((_N, _DP), jnp.float32),
        pltpu.VMEM((2, _K * _LANE), jnp.int32),
        pltpu.VMEM((2, _K * _LANE), jnp.int32),
        pltpu.VMEM((2, _K, _LANE, _DP), jnp.float32),
        pltpu.SemaphoreType.DMA,
        pltpu.SemaphoreType.DMA,
        pltpu.SemaphoreType.DMA,
    ],
)(_agg_body)


_R = 5000
_GRID = _N // _R


def _mlp_pool_body(eps_sm, x_ref, a0_ref, a1_ref, b_ref,
                   w1_ref, b1_ref, w2_ref, b2_ref, out_ref, acc_s, acc_c):
    i = pl.program_id(0)

    @pl.when(i == 0)
    def _init():
        acc_s[...] = jnp.zeros_like(acc_s)
        acc_c[...] = jnp.zeros_like(acc_c)

    h = ((1.0 + eps_sm[0]) * x_ref[...]
         + a0_ref[:, 0:_D] + a1_ref[:, 0:_D])
    h = jnp.dot(h, w1_ref[...], preferred_element_type=jnp.float32)
    h = jnp.maximum(h + b1_ref[...], 0.0)
    h = jnp.dot(h, w2_ref[...], preferred_element_type=jnp.float32)
    h = jnp.maximum(h + b2_ref[...], 0.0)

    seg = b_ref[0, 0, :]
    oh = (seg[:, None] ==
          lax.broadcasted_iota(jnp.int32, (_R, _G), 1)).astype(jnp.float32)
    acc_s[...] += lax.dot_general(oh, h, (((0,), (0,)), ((), ())),
                                  preferred_element_type=jnp.float32)
    acc_c[...] += jnp.sum(oh, axis=0, keepdims=True)

    @pl.when(i == _GRID - 1)
    def _fin():
        cnt = jnp.maximum(acc_c[0, :], 1.0)
        pooled = acc_s[...] / cnt[:, None]
        m = jnp.max(pooled, axis=1, keepdims=True)
        e = jnp.exp(pooled - m)
        lse = jnp.log(jnp.sum(e, axis=1, keepdims=True))
        out_ref[...] = pooled - m - lse


def _mlp_pool(x, aggflat, batch, eps, W1, b1, W2, b2):
    batch3 = batch.reshape(_GRID, 1, _R)
    return pl.pallas_call(
        _mlp_pool_body,
        grid=(_GRID,),
        in_specs=[
            pl.BlockSpec(memory_space=pltpu.SMEM),
            pl.BlockSpec((_R, _D), lambda i: (i, 0)),
            pl.BlockSpec((_R, _DP), lambda i: (i, 0)),
            pl.BlockSpec((_R, _DP), lambda i: (i + _GRID, 0)),
            pl.BlockSpec((1, 1, _R), lambda i: (i, 0, 0)),
            pl.BlockSpec((_D, _H), lambda i: (0, 0)),
            pl.BlockSpec((1, _H), lambda i: (0, 0)),
            pl.BlockSpec((_H, _D), lambda i: (0, 0)),
            pl.BlockSpec((1, _D), lambda i: (0, 0)),
        ],
        out_specs=pl.BlockSpec((_G, _D), lambda i: (0, 0)),
        out_shape=jax.ShapeDtypeStruct((_G, _D), jnp.float32),
        scratch_shapes=[
            pltpu.VMEM((_G, _D), jnp.float32),
            pltpu.VMEM((1, _G), jnp.float32),
        ],
    )(eps.reshape(1), x, aggflat, aggflat, batch3,
      W1, b1.reshape(1, _H), W2, b2.reshape(1, _D))


def kernel(x, edge_index, batch, eps, W1, b1, W2, b2):
    xz = jnp.pad(x, ((0, _N), (0, _DP - _D)))
    aggflat = _agg_kernel(xz, edge_index)
    return _mlp_pool(x, aggflat, batch, eps, W1, b1, W2, b2)
